# baseline (speedup 1.0000x reference)
import functools
import jax
import jax.numpy as jnp
from jax import lax
from jax.experimental import pallas as pl
from jax.experimental.pallas import tpu as pltpu
from jax.experimental.pallas import tpu_sc as plsc

N_DENSE = 13
N_SPARSE = 26
VOCAB = 100000
EMBED_DIM = 16
BATCH = 16384

FPB = 8
FB = 4
VC2 = 2048
NJ2 = 49
TROWS = NJ2 * VC2 * FPB

NC = 2
NS = 16
NW = NC * NS
B_PER_W = BATCH // NW
SEG = 128
EROWS = B_PER_W * FPB
ESEG = EROWS // SEG
WSEG = B_PER_W * N_SPARSE // SEG

_MESH = plsc.VectorSubcoreMesh(core_axis_name="c", subcore_axis_name="s")
_SC_PARAMS = pltpu.CompilerParams(use_tc_tiling_on_sc=False)


def _tc_detile_fb(a2, fb):
  def body(in_ref, out_ref):
    out_ref[...] = in_ref[...].T

  return pl.pallas_call(
      body,
      grid=(NJ2,),
      in_specs=[pl.BlockSpec((FPB * EMBED_DIM, VC2), lambda j: (fb, j))],
      out_specs=pl.BlockSpec((VC2, FPB * EMBED_DIM), lambda j: (j, 0)),
      out_shape=jax.ShapeDtypeStruct((NJ2 * VC2, FPB * EMBED_DIM),
                                     jnp.float32),
  )(a2)


def _sc_gather_emb(emb_flat, idx2d):
  @functools.partial(
      pl.kernel,
      mesh=_MESH,
      compiler_params=_SC_PARAMS,
      out_type=jax.ShapeDtypeStruct((BATCH * FPB, EMBED_DIM), jnp.float32),
      scratch_types=[
          pltpu.VMEM((ESEG, SEG), jnp.int32),
          pltpu.VMEM((EROWS, EMBED_DIM), jnp.float32),
          pltpu.SemaphoreType.DMA,
      ],
  )
  def k(emb_hbm, idx_hbm, h_out, idx_v, emb_v, sem_e):
    wid = lax.axis_index("s") * NC + lax.axis_index("c")
    pltpu.sync_copy(idx_hbm.at[pl.ds(wid * ESEG, ESEG)], idx_v)
    cps = []
    for j in range(ESEG):
      cps.append(pltpu.async_copy(
          emb_hbm.at[idx_v.at[j]], emb_v.at[pl.ds(j * SEG, SEG)], sem_e))
    for cp in cps:
      cp.wait()
    pltpu.sync_copy(emb_v, h_out.at[pl.ds(wid * EROWS, EROWS)])

  return k(emb_flat, idx2d)


def _sc_gather_wide(wide_flat, idx2dw):
  @functools.partial(
      pl.kernel,
      mesh=_MESH,
      compiler_params=_SC_PARAMS,
      out_type=jax.ShapeDtypeStruct((BATCH,), jnp.float32),
      scratch_types=[
          pltpu.VMEM((WSEG, SEG), jnp.int32),
          pltpu.VMEM((WSEG * SEG,), jnp.float32),
          pltpu.VMEM((B_PER_W,), jnp.float32),
          pltpu.SemaphoreType.DMA,
      ],
  )
  def k(wide_hbm, idxw_hbm, wsum_out, idxw_v, wval_v, wsum_v, sem_w):
    wid = lax.axis_index("s") * NC + lax.axis_index("c")
    pltpu.sync_copy(idxw_hbm.at[pl.ds(wid * WSEG, WSEG)], idxw_v)
    cps = []
    for j in range(WSEG):
      cps.append(pltpu.async_copy(
          wide_hbm.at[idxw_v.at[j]], wval_v.at[pl.ds(j * SEG, SEG)], sem_w))
    for cp in cps:
      cp.wait()
    for jc in range(B_PER_W // 16):
      acc = wval_v[pl.ds(jc * 16, 16)]
      for f in range(1, N_SPARSE):
        acc = acc + wval_v[pl.ds(f * B_PER_W + jc * 16, 16)]
      wsum_v[pl.ds(jc * 16, 16)] = acc
    pltpu.sync_copy(wsum_v, wsum_out.at[pl.ds(wid * B_PER_W, B_PER_W)])

  return k(wide_flat, idx2dw)


def _tc_mlp(hs, xin, wsum, wd_pad, wb, W0p, b0, W1, b1, W2, b2, w3r, b3):
  BLK = 2048
  grid = (BATCH // BLK,)

  def body(h0_ref, h1_ref, h2_ref, h3_ref, x_ref, ws_ref, wd_ref, wb_ref,
           W0_ref, b0_ref, W1_ref, b1_ref, W2_ref, b2_ref, w3_ref, b3_ref,
           o_ref):
    hrefs = (h0_ref, h1_ref, h2_ref, h3_ref)
    a = jnp.dot(hrefs[0][...], W0_ref[:128],
                preferred_element_type=jnp.float32)
    for fb in range(1, FB):
      a = a + jnp.dot(hrefs[fb][...], W0_ref[fb * 128:(fb + 1) * 128],
                      preferred_element_type=jnp.float32)
    a = jnp.maximum(a + b0_ref[...], 0.0)
    a = jnp.dot(a, W1_ref[...], preferred_element_type=jnp.float32)
    a = jnp.maximum(a + b1_ref[...], 0.0)
    a = jnp.dot(a, W2_ref[...], preferred_element_type=jnp.float32)
    a = jnp.maximum(a + b2_ref[...], 0.0)
    deep = jnp.sum(a * w3_ref[...], axis=1, keepdims=True) + b3_ref[...]
    dense = jnp.sum(x_ref[...] * wd_ref[...], axis=1, keepdims=True)
    wide = dense + ws_ref[...] + wb_ref[...]
    o_ref[...] = jax.nn.sigmoid(0.5 * (wide + deep))

  full = lambda shape: pl.BlockSpec(shape, lambda i: (0, 0))
  hspec = pl.BlockSpec((BLK, FPB * EMBED_DIM), lambda i: (i, 0))
  return pl.pallas_call(
      body,
      grid=grid,
      in_specs=[
          hspec, hspec, hspec, hspec,
          pl.BlockSpec((BLK, N_DENSE + N_SPARSE), lambda i: (i, 0)),
          pl.BlockSpec((BLK, 1), lambda i: (i, 0)),
          full(wd_pad.shape),
          full(wb.shape),
          full(W0p.shape),
          full(b0.shape),
          full(W1.shape),
          full(b1.shape),
          full(W2.shape),
          full(b2.shape),
          full(w3r.shape),
          full(b3.shape),
      ],
      out_specs=pl.BlockSpec((BLK, 1), lambda i: (i, 0)),
      out_shape=jax.ShapeDtypeStruct((BATCH, 1), jnp.float32),
  )(*hs, xin, wsum, wd_pad, wb, W0p, b0, W1, b1, W2, b2, w3r, b3)


def kernel(inputs, embed_tables, wide_tables, w_dense, wide_b,
           W0, b0, W1, b1, W2, b2, W3, b3):
  sparse = inputs[:, N_DENSE:].astype(jnp.int32)
  farange = jnp.arange(N_SPARSE, dtype=jnp.int32)
  offsw = (farange * VOCAB)[None, :]
  idx2dw = (sparse + offsw).reshape(NW, B_PER_W, N_SPARSE).transpose(
      0, 2, 1).reshape(NW * WSEG, SEG)

  v32 = jnp.concatenate(
      [sparse, jnp.tile(sparse[:, 24:25], (1, FB * FPB - N_SPARSE))], axis=1)
  fi32 = (jnp.arange(FB * FPB, dtype=jnp.int32) % FPB)[None, :]
  r32 = (v32 // VC2) * (VC2 * 8) + (v32 % VC2) * 8 + fi32
  idx_all = r32.reshape(BATCH, FB, FPB).transpose(1, 0, 2).reshape(
      FB, NW * ESEG, SEG)
  idx_fbs = [idx_all[fb] for fb in range(FB)]

  a2 = jnp.transpose(embed_tables, (0, 2, 1)).reshape(
      N_SPARSE * EMBED_DIM, VOCAB)
  wide_flat = wide_tables.reshape(-1)

  wsum = _sc_gather_wide(wide_flat, idx2dw)
  hs = []
  for fb in range(FB):
    tbl = _tc_detile_fb(a2, fb).reshape(TROWS, EMBED_DIM)
    hfb = _sc_gather_emb(tbl, idx_fbs[fb])
    hs.append(hfb.reshape(BATCH, FPB * EMBED_DIM))

  W0p = jnp.concatenate(
      [W0, jnp.zeros((FB * FPB * EMBED_DIM - W0.shape[0], W0.shape[1]),
                     jnp.float32)], axis=0)
  wd_pad = jnp.concatenate(
      [w_dense[:, 0], jnp.zeros((N_SPARSE,), jnp.float32)])[None, :]
  return _tc_mlp(hs, inputs, wsum.reshape(BATCH, 1), wd_pad,
                 wide_b.reshape(1, 1), W0p, b0[None, :], W1, b1[None, :],
                 W2, b2[None, :], W3.reshape(1, -1), b3.reshape(1, 1))


# speedup vs baseline: 1.1070x; 1.1070x over previous
import functools
import jax
import jax.numpy as jnp
from jax import lax
from jax.experimental import pallas as pl
from jax.experimental.pallas import tpu as pltpu
from jax.experimental.pallas import tpu_sc as plsc

N_DENSE = 13
N_SPARSE = 26
VOCAB = 100000
EMBED_DIM = 16
BATCH = 16384

FPB = 8
FB = 4
VC2 = 2048
NJ2 = 49
TROWS = NJ2 * VC2 * FPB
VC4 = 8192
NJ4 = 13

NC = 2
NS = 16
NW = NC * NS
B_PER_W = BATCH // NW
SEG = 128
EROWS = B_PER_W * FPB
ESEG = EROWS // SEG
WSEG = B_PER_W * N_SPARSE // SEG

def _mesh():
  return plsc.VectorSubcoreMesh(core_axis_name="c", subcore_axis_name="s")


_SC_PARAMS = pltpu.CompilerParams(use_tc_tiling_on_sc=False)


def _tc_detile_fb(a2, fb):
  def body(in_ref, out_ref):
    out_ref[...] = in_ref[...].T

  return pl.pallas_call(
      body,
      grid=(NJ2,),
      in_specs=[pl.BlockSpec((FPB * EMBED_DIM, VC2), lambda j: (fb, j))],
      out_specs=pl.BlockSpec((VC2, FPB * EMBED_DIM), lambda j: (j, 0)),
      out_shape=jax.ShapeDtypeStruct((NJ2 * VC2, FPB * EMBED_DIM),
                                     jnp.float32),
  )(a2)


def _tc_detile_fb3(a2):
  def body(in_ref, out_ref):
    x = in_ref[...]
    for q in range(4):
      out_ref[:, q * 32:(q + 1) * 32] = x[:, q * VC2:(q + 1) * VC2].T

  return pl.pallas_call(
      body,
      grid=(NJ4,),
      in_specs=[pl.BlockSpec((2 * EMBED_DIM, VC4), lambda j: (12, j))],
      out_specs=pl.BlockSpec((VC2, FPB * EMBED_DIM), lambda j: (j, 0)),
      out_shape=jax.ShapeDtypeStruct((NJ4 * VC2, FPB * EMBED_DIM),
                                     jnp.float32),
  )(a2)


def _sc_gather_emb(emb_flat, idx2d):
  @functools.partial(
      pl.kernel,
      mesh=_mesh(),
      compiler_params=_SC_PARAMS,
      out_type=jax.ShapeDtypeStruct((BATCH * FPB, EMBED_DIM), jnp.float32),
      scratch_types=[
          pltpu.VMEM((ESEG, SEG), jnp.int32),
          pltpu.VMEM((EROWS, EMBED_DIM), jnp.float32),
          pltpu.SemaphoreType.DMA,
      ],
  )
  def k(emb_hbm, idx_hbm, h_out, idx_v, emb_v, sem_e):
    wid = lax.axis_index("s") * NC + lax.axis_index("c")
    pltpu.sync_copy(idx_hbm.at[pl.ds(wid * ESEG, ESEG)], idx_v)
    cps = []
    for j in range(ESEG):
      cps.append(pltpu.async_copy(
          emb_hbm.at[idx_v.at[j]], emb_v.at[pl.ds(j * SEG, SEG)], sem_e))
    for cp in cps:
      cp.wait()
    pltpu.sync_copy(emb_v, h_out.at[pl.ds(wid * EROWS, EROWS)])

  return k(emb_flat, idx2d)


def _sc_gather_wide(wide_flat, idx2dw):
  @functools.partial(
      pl.kernel,
      mesh=_mesh(),
      compiler_params=_SC_PARAMS,
      out_type=jax.ShapeDtypeStruct((BATCH * N_SPARSE,), jnp.float32),
      scratch_types=[
          pltpu.VMEM((WSEG, SEG), jnp.int32),
          pltpu.VMEM((WSEG * SEG,), jnp.float32),
          pltpu.SemaphoreType.DMA,
      ],
  )
  def k(wide_hbm, idxw_hbm, wval_out, idxw_v, wval_v, sem_w):
    wid = lax.axis_index("s") * NC + lax.axis_index("c")
    pltpu.sync_copy(idxw_hbm.at[pl.ds(wid * WSEG, WSEG)], idxw_v)
    cps = []
    for j in range(WSEG):
      cps.append(pltpu.async_copy(
          wide_hbm.at[idxw_v.at[j]], wval_v.at[pl.ds(j * SEG, SEG)], sem_w))
    for cp in cps:
      cp.wait()
    pltpu.sync_copy(wval_v, wval_out.at[pl.ds(wid * WSEG * SEG, WSEG * SEG)])

  return k(wide_flat, idx2dw)


def _tc_mlp(hs, xin, wval, wd_pad, wb, W0p, b0, W1, b1, W2, b2, w3r, b3):
  BLK = 2048
  grid = (BATCH // BLK,)

  def body(h0_ref, h1_ref, h2_ref, h3_ref, x_ref, wv_ref, wd_ref, wb_ref,
           W0_ref, b0_ref, W1_ref, b1_ref, W2_ref, b2_ref, w3_ref, b3_ref,
           o_ref):
    hrefs = (h0_ref, h1_ref, h2_ref, h3_ref)
    a = jnp.dot(hrefs[0][...], W0_ref[:128],
                preferred_element_type=jnp.float32)
    for fb in range(1, FB):
      a = a + jnp.dot(hrefs[fb][...], W0_ref[fb * 128:(fb + 1) * 128],
                      preferred_element_type=jnp.float32)
    a = jnp.maximum(a + b0_ref[...], 0.0)
    a = jnp.dot(a, W1_ref[...], preferred_element_type=jnp.float32)
    a = jnp.maximum(a + b1_ref[...], 0.0)
    a = jnp.dot(a, W2_ref[...], preferred_element_type=jnp.float32)
    a = jnp.maximum(a + b2_ref[...], 0.0)
    deep = jnp.sum(a * w3_ref[...], axis=1, keepdims=True) + b3_ref[...]
    dense = jnp.sum(x_ref[...] * wd_ref[...], axis=1, keepdims=True)
    wsum = jnp.sum(wv_ref[...], axis=1, keepdims=True)
    wide = dense + wsum + wb_ref[...]
    o_ref[...] = jax.nn.sigmoid(0.5 * (wide + deep))

  full = lambda shape: pl.BlockSpec(shape, lambda i: (0, 0))
  hspec = pl.BlockSpec((BLK, FPB * EMBED_DIM), lambda i: (i, 0))
  return pl.pallas_call(
      body,
      grid=grid,
      in_specs=[
          hspec, hspec, hspec, hspec,
          pl.BlockSpec((BLK, N_DENSE + N_SPARSE), lambda i: (i, 0)),
          pl.BlockSpec((BLK, N_SPARSE), lambda i: (i, 0)),
          full(wd_pad.shape),
          full(wb.shape),
          full(W0p.shape),
          full(b0.shape),
          full(W1.shape),
          full(b1.shape),
          full(W2.shape),
          full(b2.shape),
          full(w3r.shape),
          full(b3.shape),
      ],
      out_specs=pl.BlockSpec((BLK, 1), lambda i: (i, 0)),
      out_shape=jax.ShapeDtypeStruct((BATCH, 1), jnp.float32),
  )(*hs, xin, wval, wd_pad, wb, W0p, b0, W1, b1, W2, b2, w3r, b3)


def kernel(inputs, embed_tables, wide_tables, w_dense, wide_b,
           W0, b0, W1, b1, W2, b2, W3, b3):
  sparse = inputs[:, N_DENSE:].astype(jnp.int32)
  farange = jnp.arange(N_SPARSE, dtype=jnp.int32)
  offsw = (farange * VOCAB)[None, :]
  idx2dw = (sparse + offsw).reshape(NW * WSEG, SEG)

  idx_fbs = []
  for fb in range(FB - 1):
    v = sparse[:, fb * FPB:(fb + 1) * FPB]
    fi = jnp.arange(FPB, dtype=jnp.int32)[None, :]
    r = (v // VC2) * (VC2 * 8) + (v % VC2) * 8 + fi
    idx_fbs.append(r.reshape(NW * ESEG, SEG))
  v3 = sparse[:, 24:26]
  fi = jnp.arange(2, dtype=jnp.int32)[None, :]
  r3 = ((v3 // VC4) * (VC2 * 8) + (v3 % VC2) * 8
        + ((v3 % VC4) // VC2) * 2 + fi)
  r3 = jnp.concatenate([r3] + [r3[:, :1]] * (FPB - 2), axis=1)
  idx_fbs.append(r3.reshape(NW * ESEG, SEG))

  a2 = jnp.transpose(embed_tables, (0, 2, 1)).reshape(
      N_SPARSE * EMBED_DIM, VOCAB)
  wide_flat = wide_tables.reshape(-1)

  wval = _sc_gather_wide(wide_flat, idx2dw)
  hs = []
  for fb in range(FB - 1):
    tbl = _tc_detile_fb(a2, fb).reshape(TROWS, EMBED_DIM)
    hfb = _sc_gather_emb(tbl, idx_fbs[fb])
    hs.append(hfb.reshape(BATCH, FPB * EMBED_DIM))
  tbl3 = _tc_detile_fb3(a2).reshape(NJ4 * VC2 * 8, EMBED_DIM)
  hfb3 = _sc_gather_emb(tbl3, idx_fbs[FB - 1])
  hs.append(hfb3.reshape(BATCH, FPB * EMBED_DIM))

  W0p = jnp.concatenate(
      [W0, jnp.zeros((FB * FPB * EMBED_DIM - W0.shape[0], W0.shape[1]),
                     jnp.float32)], axis=0)
  wd_pad = jnp.concatenate(
      [w_dense[:, 0], jnp.zeros((N_SPARSE,), jnp.float32)])[None, :]
  return _tc_mlp(hs, inputs, wval.reshape(BATCH, N_SPARSE), wd_pad,
                 wide_b.reshape(1, 1), W0p, b0[None, :], W1, b1[None, :],
                 W2, b2[None, :], W3.reshape(1, -1), b3.reshape(1, 1))


# speedup vs baseline: 1.1099x; 1.0026x over previous
import functools
import jax
import jax.numpy as jnp
from jax import lax
from jax.experimental import pallas as pl
from jax.experimental.pallas import tpu as pltpu
from jax.experimental.pallas import tpu_sc as plsc

N_DENSE = 13
N_SPARSE = 26
VOCAB = 100000
EMBED_DIM = 16
BATCH = 16384

FPB = 8
FB = 4
VC2 = 2048
NJ2 = 49
TROWS = NJ2 * VC2 * FPB
VC4 = 8192
NJ4 = 13

NC = 2
NS = 16
NW = NC * NS
B_PER_W = BATCH // NW
SEG = 128
EROWS = B_PER_W * FPB
ESEG = EROWS // SEG
WSEG = B_PER_W * N_SPARSE // SEG

def _mesh():
  return plsc.VectorSubcoreMesh(core_axis_name="c", subcore_axis_name="s")


_SC_PARAMS = pltpu.CompilerParams(use_tc_tiling_on_sc=False)


def _tc_detile_fb(a2, fb):
  def body(in_ref, out_ref):
    out_ref[...] = in_ref[...].T

  return pl.pallas_call(
      body,
      grid=(NJ2,),
      in_specs=[pl.BlockSpec((FPB * EMBED_DIM, VC2), lambda j: (fb, j))],
      out_specs=pl.BlockSpec((VC2, FPB * EMBED_DIM), lambda j: (j, 0)),
      out_shape=jax.ShapeDtypeStruct((NJ2 * VC2, FPB * EMBED_DIM),
                                     jnp.float32),
  )(a2)


def _tc_detile_fb3(a2):
  def body(in_ref, out_ref):
    x = in_ref[...]
    for q in range(4):
      out_ref[:, q * 32:(q + 1) * 32] = x[:, q * VC2:(q + 1) * VC2].T

  return pl.pallas_call(
      body,
      grid=(NJ4,),
      in_specs=[pl.BlockSpec((2 * EMBED_DIM, VC4), lambda j: (12, j))],
      out_specs=pl.BlockSpec((VC2, FPB * EMBED_DIM), lambda j: (j, 0)),
      out_shape=jax.ShapeDtypeStruct((NJ4 * VC2, FPB * EMBED_DIM),
                                     jnp.float32),
  )(a2)


def _sc_gather_emb(emb_flat, idx2d):
  @functools.partial(
      pl.kernel,
      mesh=_mesh(),
      compiler_params=_SC_PARAMS,
      out_type=jax.ShapeDtypeStruct((BATCH * FPB, EMBED_DIM), jnp.float32),
      scratch_types=[
          pltpu.VMEM((ESEG, SEG), jnp.int32),
          pltpu.VMEM((EROWS, EMBED_DIM), jnp.float32),
          pltpu.SemaphoreType.DMA,
      ],
  )
  def k(emb_hbm, idx_hbm, h_out, idx_v, emb_v, sem_e):
    wid = lax.axis_index("s") * NC + lax.axis_index("c")
    pltpu.sync_copy(idx_hbm.at[pl.ds(wid * ESEG, ESEG)], idx_v)
    cps = []
    for j in range(ESEG):
      cps.append(pltpu.async_copy(
          emb_hbm.at[idx_v.at[j]], emb_v.at[pl.ds(j * SEG, SEG)], sem_e))
    for cp in cps:
      cp.wait()
    pltpu.sync_copy(emb_v, h_out.at[pl.ds(wid * EROWS, EROWS)])

  return k(emb_flat, idx2d)


def _sc_gather_wide(wide_flat, idx2dw):
  @functools.partial(
      pl.kernel,
      mesh=_mesh(),
      compiler_params=_SC_PARAMS,
      out_type=jax.ShapeDtypeStruct((BATCH * N_SPARSE,), jnp.float32),
      scratch_types=[
          pltpu.VMEM((WSEG, SEG), jnp.int32),
          pltpu.VMEM((WSEG * SEG,), jnp.float32),
          pltpu.SemaphoreType.DMA,
      ],
  )
  def k(wide_hbm, idxw_hbm, wval_out, idxw_v, wval_v, sem_w):
    wid = lax.axis_index("s") * NC + lax.axis_index("c")
    pltpu.sync_copy(idxw_hbm.at[pl.ds(wid * WSEG, WSEG)], idxw_v)
    cps = []
    for j in range(WSEG):
      cps.append(pltpu.async_copy(
          wide_hbm.at[idxw_v.at[j]], wval_v.at[pl.ds(j * SEG, SEG)], sem_w))
    for cp in cps:
      cp.wait()
    pltpu.sync_copy(wval_v, wval_out.at[pl.ds(wid * WSEG * SEG, WSEG * SEG)])

  return k(wide_flat, idx2dw)


def _tc_mlp(hs, xin, wval, wd_pad, wb, W0p, b0, W1, b1, W2, b2, w3r, b3):
  BLK = 4096
  grid = (BATCH // BLK,)

  def body(h0_ref, h1_ref, h2_ref, h3_ref, x_ref, wv_ref, wd_ref, wb_ref,
           W0_ref, b0_ref, W1_ref, b1_ref, W2_ref, b2_ref, w3_ref, b3_ref,
           o_ref):
    hrefs = (h0_ref, h1_ref, h2_ref, h3_ref)
    a = jnp.dot(hrefs[0][...], W0_ref[:128],
                preferred_element_type=jnp.float32)
    for fb in range(1, FB):
      a = a + jnp.dot(hrefs[fb][...], W0_ref[fb * 128:(fb + 1) * 128],
                      preferred_element_type=jnp.float32)
    a = jnp.maximum(a + b0_ref[...], 0.0)
    a = jnp.dot(a, W1_ref[...], preferred_element_type=jnp.float32)
    a = jnp.maximum(a + b1_ref[...], 0.0)
    a = jnp.dot(a, W2_ref[...], preferred_element_type=jnp.float32)
    a = jnp.maximum(a + b2_ref[...], 0.0)
    deep = jnp.sum(a * w3_ref[...], axis=1, keepdims=True) + b3_ref[...]
    dense = jnp.sum(x_ref[...] * wd_ref[...], axis=1, keepdims=True)
    wsum = jnp.sum(wv_ref[...], axis=1, keepdims=True)
    wide = dense + wsum + wb_ref[...]
    o_ref[...] = jax.nn.sigmoid(0.5 * (wide + deep))

  full = lambda shape: pl.BlockSpec(shape, lambda i: (0, 0))
  hspec = pl.BlockSpec((BLK, FPB * EMBED_DIM), lambda i: (i, 0))
  return pl.pallas_call(
      body,
      grid=grid,
      in_specs=[
          hspec, hspec, hspec, hspec,
          pl.BlockSpec((BLK, N_DENSE + N_SPARSE), lambda i: (i, 0)),
          pl.BlockSpec((BLK, N_SPARSE), lambda i: (i, 0)),
          full(wd_pad.shape),
          full(wb.shape),
          full(W0p.shape),
          full(b0.shape),
          full(W1.shape),
          full(b1.shape),
          full(W2.shape),
          full(b2.shape),
          full(w3r.shape),
          full(b3.shape),
      ],
      out_specs=pl.BlockSpec((BLK, 1), lambda i: (i, 0)),
      out_shape=jax.ShapeDtypeStruct((BATCH, 1), jnp.float32),
  )(*hs, xin, wval, wd_pad, wb, W0p, b0, W1, b1, W2, b2, w3r, b3)


def kernel(inputs, embed_tables, wide_tables, w_dense, wide_b,
           W0, b0, W1, b1, W2, b2, W3, b3):
  sparse = inputs[:, N_DENSE:].astype(jnp.int32)
  farange = jnp.arange(N_SPARSE, dtype=jnp.int32)
  offsw = (farange * VOCAB)[None, :]
  idx2dw = (sparse + offsw).reshape(NW * WSEG, SEG)

  idx_fbs = []
  for fb in range(FB - 1):
    v = sparse[:, fb * FPB:(fb + 1) * FPB]
    fi = jnp.arange(FPB, dtype=jnp.int32)[None, :]
    r = (v // VC2) * (VC2 * 8) + (v % VC2) * 8 + fi
    idx_fbs.append(r.reshape(NW * ESEG, SEG))
  v3 = sparse[:, 24:26]
  fi = jnp.arange(2, dtype=jnp.int32)[None, :]
  r3 = ((v3 // VC4) * (VC2 * 8) + (v3 % VC2) * 8
        + ((v3 % VC4) // VC2) * 2 + fi)
  r3 = jnp.concatenate([r3] + [r3[:, :1]] * (FPB - 2), axis=1)
  idx_fbs.append(r3.reshape(NW * ESEG, SEG))

  a2 = jnp.transpose(embed_tables, (0, 2, 1)).reshape(
      N_SPARSE * EMBED_DIM, VOCAB)
  wide_flat = wide_tables.reshape(-1)

  wval = _sc_gather_wide(wide_flat, idx2dw)
  hs = []
  for fb in range(FB - 1):
    tbl = _tc_detile_fb(a2, fb).reshape(TROWS, EMBED_DIM)
    hfb = _sc_gather_emb(tbl, idx_fbs[fb])
    hs.append(hfb.reshape(BATCH, FPB * EMBED_DIM))
  tbl3 = _tc_detile_fb3(a2).reshape(NJ4 * VC2 * 8, EMBED_DIM)
  hfb3 = _sc_gather_emb(tbl3, idx_fbs[FB - 1])
  hs.append(hfb3.reshape(BATCH, FPB * EMBED_DIM))

  W0p = jnp.concatenate(
      [W0, jnp.zeros((FB * FPB * EMBED_DIM - W0.shape[0], W0.shape[1]),
                     jnp.float32)], axis=0)
  wd_pad = jnp.concatenate(
      [w_dense[:, 0], jnp.zeros((N_SPARSE,), jnp.float32)])[None, :]
  return _tc_mlp(hs, inputs, wval.reshape(BATCH, N_SPARSE), wd_pad,
                 wide_b.reshape(1, 1), W0p, b0[None, :], W1, b1[None, :],
                 W2, b2[None, :], W3.reshape(1, -1), b3.reshape(1, 1))
